# baseline (speedup 1.0000x reference)
import functools

import jax
import jax.numpy as jnp
from jax import lax
from jax.experimental import pallas as pl
from jax.experimental.pallas import tpu as pltpu
from jax.experimental.pallas import tpu_sc as plsc

NC, NS = 2, 16
NW = NC * NS
LANES = 16
IB = 4
KL = 4


def _sc_gather_t(idsT, emb):
    L, B = idsT.shape
    D = emb.shape[1]
    BB = B // NW
    NIT = L // IB

    @functools.partial(
        pl.kernel,
        mesh=plsc.VectorSubcoreMesh(core_axis_name="c", subcore_axis_name="s"),
        out_type=jax.ShapeDtypeStruct((L * D, B), jnp.float32),
        scratch_types=[
            pltpu.VMEM((L, BB), jnp.int32),
            pltpu.VMEM((2 * IB * BB, D), jnp.float32),
            pltpu.VMEM((2, IB * D, BB), jnp.float32),
            pltpu.SemaphoreType.DMA,
            pltpu.SemaphoreType.DMA,
        ],
        compiler_params=pltpu.CompilerParams(
            use_tc_tiling_on_sc=False, needs_layout_passes=False
        ),
    )
    def k(ids_hbm, emb_hbm, out_hbm, idx_all, tiles, ttile, sem_g, sem_o):
        wid = lax.axis_index("s") * NC + lax.axis_index("c")
        b0 = wid * BB
        lane = lax.iota(jnp.int32, LANES)
        dconst = [jnp.full((LANES,), d, jnp.int32) for d in range(D)]
        pltpu.sync_copy(ids_hbm.at[:, pl.ds(b0, BB)], idx_all)

        def fire_gathers(it, buf):
            for j in range(IB):
                pltpu.async_copy(
                    emb_hbm.at[idx_all.at[it * IB + j]],
                    tiles.at[pl.ds((buf * IB + j) * BB, BB)],
                    sem_g,
                )

        def drain_gather():
            pltpu.make_async_copy(
                emb_hbm.at[pl.ds(0, BB)], tiles.at[pl.ds(0, BB)], sem_g
            ).wait()

        def drain_out():
            pltpu.make_async_copy(
                out_hbm.at[pl.ds(0, IB * D), pl.ds(0, BB)],
                ttile.at[0],
                sem_o,
            ).wait()

        fire_gathers(0, 0)

        def outer(i2, carry):
            for ph in range(2):
                it = i2 * 2 + ph
                cur, nxt = ph, 1 - ph
                for _ in range(IB):
                    drain_gather()

                @pl.when(it + 1 < NIT)
                def _():
                    fire_gathers(it + 1, nxt)

                @pl.when(it >= 2)
                def _():
                    drain_out()

                for j in range(IB):

                    def cbody(c, c2, cur=cur, j=j):
                        rowv = lane + (c * LANES + (cur * IB + j) * BB)
                        for d in range(D):
                            v = plsc.load_gather(tiles, [rowv, dconst[d]])
                            ttile[cur, j * D + d, pl.ds(c * LANES, LANES)] = v
                        return c2

                    lax.fori_loop(0, BB // LANES, cbody, 0)
                pltpu.async_copy(
                    ttile.at[cur],
                    out_hbm.at[pl.ds(it * IB * D, IB * D), pl.ds(b0, BB)],
                    sem_o,
                )
            return carry

        lax.fori_loop(0, NIT // 2, outer, 0)
        drain_out()
        drain_out()

    return k(idsT, emb)


def _tc_project(x2, wk, bk):
    M, B = x2.shape

    def body(x_ref, w_ref, b_ref, o_ref):
        o_ref[...] = (
            jnp.dot(w_ref[...], x_ref[...], preferred_element_type=jnp.float32)
            + b_ref[...]
        )

    return pl.pallas_call(
        body,
        grid=(M // wk.shape[0],),
        in_specs=[
            pl.BlockSpec((wk.shape[0], B), lambda i: (i, 0)),
            pl.BlockSpec(wk.shape, lambda i: (0, 0)),
            pl.BlockSpec((wk.shape[0], 1), lambda i: (0, 0)),
        ],
        out_specs=pl.BlockSpec((wk.shape[0], B), lambda i: (i, 0)),
        out_shape=jax.ShapeDtypeStruct((M, B), jnp.float32),
    )(x2, wk, bk)


def kernel(input_ids, emb, W, b):
    Bt, L = input_ids.shape
    V, D = emb.shape
    idsT = input_ids.T.astype(jnp.int32)
    x2 = _sc_gather_t(idsT, emb)
    wk = jnp.kron(jnp.eye(KL, dtype=W.dtype), W)
    bk = jnp.tile(b, KL).reshape(KL * D, 1)
    y2 = _tc_project(x2, wk, bk)
    yT = y2.reshape(L, D, Bt)
    return jnp.transpose(yT, (2, 0, 1))


# speedup vs baseline: 1.1404x; 1.1404x over previous
import functools

import jax
import jax.numpy as jnp
from jax import lax
from jax.experimental import pallas as pl
from jax.experimental.pallas import tpu as pltpu
from jax.experimental.pallas import tpu_sc as plsc

NC, NS = 2, 16
NW = NC * NS
LANES = 16
IB = 4
KL = 4


def _sc_gather_t(idsT, emb):
    L, B = idsT.shape
    D = emb.shape[1]
    BB = B // NW
    NIT = L // IB

    @functools.partial(
        pl.kernel,
        mesh=plsc.VectorSubcoreMesh(core_axis_name="c", subcore_axis_name="s"),
        out_type=jax.ShapeDtypeStruct((L * D, B), jnp.float32),
        scratch_types=[
            pltpu.VMEM((L, BB), jnp.int32),
            pltpu.VMEM((2 * IB * BB, D), jnp.float32),
            pltpu.VMEM((BB, D + 1), jnp.float32),
            pltpu.VMEM((2, IB * D, BB), jnp.float32),
            pltpu.SemaphoreType.DMA,
            pltpu.SemaphoreType.DMA,
        ],
        compiler_params=pltpu.CompilerParams(
            use_tc_tiling_on_sc=False, needs_layout_passes=False
        ),
    )
    def k(ids_hbm, emb_hbm, out_hbm, idx_all, tiles, tilep, ttile, sem_g, sem_o):
        wid = lax.axis_index("s") * NC + lax.axis_index("c")
        b0 = wid * BB
        lane = lax.iota(jnp.int32, LANES)
        dconst = [jnp.full((LANES,), d, jnp.int32) for d in range(D)]
        pltpu.sync_copy(ids_hbm.at[:, pl.ds(b0, BB)], idx_all)

        def fire_gathers(it, buf):
            for j in range(IB):
                pltpu.async_copy(
                    emb_hbm.at[idx_all.at[it * IB + j]],
                    tiles.at[pl.ds((buf * IB + j) * BB, BB)],
                    sem_g,
                )

        def drain_gather():
            pltpu.make_async_copy(
                emb_hbm.at[pl.ds(0, BB)], tiles.at[pl.ds(0, BB)], sem_g
            ).wait()

        def drain_out():
            pltpu.make_async_copy(
                out_hbm.at[pl.ds(0, IB * D), pl.ds(0, BB)],
                ttile.at[0],
                sem_o,
            ).wait()

        fire_gathers(0, 0)

        def outer(i2, carry):
            for ph in range(2):
                it = i2 * 2 + ph
                cur, nxt = ph, 1 - ph
                for _ in range(IB):
                    drain_gather()

                @pl.when(it + 1 < NIT)
                def _():
                    fire_gathers(it + 1, nxt)

                @pl.when(it >= 2)
                def _():
                    drain_out()

                for j in range(IB):
                    base = (cur * IB + j) * BB

                    def pbody(c, c2, base=base):
                        for rr in range(LANES):
                            r = c * LANES + rr
                            tilep[r, pl.ds(0, LANES)] = tiles[
                                base + r, pl.ds(0, LANES)
                            ]
                            tilep[r, pl.ds(LANES, LANES)] = tiles[
                                base + r, pl.ds(LANES, LANES)
                            ]
                        return c2

                    lax.fori_loop(0, BB // LANES, pbody, 0)

                    def cbody(c, c2, cur=cur, j=j):
                        rowv = lane + c * LANES
                        for d in range(D):
                            v = plsc.load_gather(tilep, [rowv, dconst[d]])
                            ttile[cur, j * D + d, pl.ds(c * LANES, LANES)] = v
                        return c2

                    lax.fori_loop(0, BB // LANES, cbody, 0)
                pltpu.async_copy(
                    ttile.at[cur],
                    out_hbm.at[pl.ds(it * IB * D, IB * D), pl.ds(b0, BB)],
                    sem_o,
                )
            return carry

        lax.fori_loop(0, NIT // 2, outer, 0)
        drain_out()
        drain_out()

    return k(idsT, emb)


def _tc_project(x2, wk, bk):
    M, B = x2.shape

    def body(x_ref, w_ref, b_ref, o_ref):
        o_ref[...] = (
            jnp.dot(w_ref[...], x_ref[...], preferred_element_type=jnp.float32)
            + b_ref[...]
        )

    return pl.pallas_call(
        body,
        grid=(M // wk.shape[0],),
        in_specs=[
            pl.BlockSpec((wk.shape[0], B), lambda i: (i, 0)),
            pl.BlockSpec(wk.shape, lambda i: (0, 0)),
            pl.BlockSpec((wk.shape[0], 1), lambda i: (0, 0)),
        ],
        out_specs=pl.BlockSpec((wk.shape[0], B), lambda i: (i, 0)),
        out_shape=jax.ShapeDtypeStruct((M, B), jnp.float32),
    )(x2, wk, bk)


def kernel(input_ids, emb, W, b):
    Bt, L = input_ids.shape
    V, D = emb.shape
    idsT = input_ids.T.astype(jnp.int32)
    x2 = _sc_gather_t(idsT, emb)
    wk = jnp.kron(jnp.eye(KL, dtype=W.dtype), W)
    bk = jnp.tile(b, KL).reshape(KL * D, 1)
    y2 = _tc_project(x2, wk, bk)
    yT = y2.reshape(L, D, Bt)
    return jnp.transpose(yT, (2, 0, 1))
